# Initial kernel scaffold; baseline (speedup 1.0000x reference)
import jax
import jax.numpy as jnp
from jax.experimental import pallas as pl


def kernel(word_idx_tensor, pos_idx_tensor, true_tree_heads):
    raise NotImplementedError("write your pallas kernel here")


# speedup vs baseline: 1.6353x; 1.6353x over previous
import jax
import jax.numpy as jnp
from jax.experimental import pallas as pl


def _copy_body(w_ref, p_ref, t_ref, wo_ref, po_ref, to_ref):
    wo_ref[...] = w_ref[...]
    po_ref[...] = p_ref[...]
    to_ref[...] = t_ref[...]


def kernel(word_idx_tensor, pos_idx_tensor, true_tree_heads):
    out_shape = tuple(
        jax.ShapeDtypeStruct(x.shape, x.dtype)
        for x in (word_idx_tensor, pos_idx_tensor, true_tree_heads)
    )
    return pl.pallas_call(_copy_body, out_shape=out_shape)(
        word_idx_tensor, pos_idx_tensor, true_tree_heads
    )
